# baseline (speedup 1.0000x reference)
import functools

import jax
import jax.numpy as jnp
from jax import lax
from jax.experimental import pallas as pl
from jax.experimental.pallas import tpu as pltpu
from jax.experimental.pallas import tpu_sc as plsc

N = 10000
E = 320000
D = 128
EF = 16
TASK = 2

NTILES = 32
NPAD = 10016
ROWS_PER_TILE = NPAD // 16
EDGES_PER_TILE = E // NTILES
K = 48
CAP = 10080
NCH = CAP // K
SRCB = 2000
RB = 400
CB2 = 400

_SC_PARAMS = pltpu.CompilerParams(use_tc_tiling_on_sc=False,
                                  needs_layout_passes=False)


def _c_body(ef_ref, w0_ref, w1_ref, c0_ref, c1_ref):
    e = ef_ref[...]
    c0_ref[...] = jnp.dot(e, w0_ref[...], preferred_element_type=jnp.float32)
    c1_ref[...] = jnp.dot(e, w1_ref[...], preferred_element_type=jnp.float32)


_c_call = pl.pallas_call(
    _c_body,
    grid=(E // 8 // CB2,),
    in_specs=[pl.BlockSpec((CB2, 8 * EF), lambda i: (i, 0)),
              pl.BlockSpec((8 * EF, 8 * D), lambda i: (0, 0)),
              pl.BlockSpec((8 * EF, 8 * D), lambda i: (0, 0))],
    out_specs=[pl.BlockSpec((CB2, 8 * D), lambda i: (i, 0)),
               pl.BlockSpec((CB2, 8 * D), lambda i: (i, 0))],
    out_shape=[jax.ShapeDtypeStruct((E // 8, 8 * D), jnp.float32),
               jax.ShapeDtypeStruct((E // 8, 8 * D), jnp.float32)],
)


def _ab_body(h_ref, wa_ref, wb_ref, a_ref, b_ref):
    h = h_ref[...]
    a_ref[...] = jnp.dot(h, wa_ref[...], preferred_element_type=jnp.float32)
    b_ref[...] = jnp.dot(h, wb_ref[...], preferred_element_type=jnp.float32)


_ab_call = pl.pallas_call(
    _ab_body,
    grid=(N // RB,),
    in_specs=[pl.BlockSpec((RB, D), lambda i: (i, 0)),
              pl.BlockSpec((D, D), lambda i: (0, 0)),
              pl.BlockSpec((D, D), lambda i: (0, 0))],
    out_specs=[pl.BlockSpec((RB, D), lambda i: (i, 0)),
               pl.BlockSpec((RB, D), lambda i: (i, 0))],
    out_shape=[jax.ShapeDtypeStruct((N, D), jnp.float32),
               jax.ShapeDtypeStruct((N, D), jnp.float32)],
)


def _node_update(h, red_ref, recv_ref, wh_ref, wr_ref):
    red = red_ref[0] + red_ref[1]
    x = (jnp.dot(h, wh_ref[...], preferred_element_type=jnp.float32)
         + jnp.dot(red, wr_ref[...], preferred_element_type=jnp.float32))
    o = jnp.maximum(x, x * 0.01)
    cnt = recv_ref[0, :, 0] + recv_ref[1, :, 0]
    return jnp.where((cnt > 0.0)[:, None], o, 0.0)


def _update_body(h_ref, red_ref, recv_ref, wh_ref, wr_ref, out_ref):
    out_ref[...] = _node_update(h_ref[...], red_ref, recv_ref, wh_ref, wr_ref)


def _update_ab_body(h_ref, red_ref, recv_ref, wh_ref, wr_ref, wa_ref, wb_ref,
                    out_ref, a_ref, b_ref):
    h1 = _node_update(h_ref[...], red_ref, recv_ref, wh_ref, wr_ref)
    out_ref[...] = h1
    a_ref[...] = jnp.dot(h1, wa_ref[...], preferred_element_type=jnp.float32)
    b_ref[...] = jnp.dot(h1, wb_ref[...], preferred_element_type=jnp.float32)


_update_call = pl.pallas_call(
    _update_body,
    grid=(N // RB,),
    in_specs=[pl.BlockSpec((RB, D), lambda i: (i, 0)),
              pl.BlockSpec((2, RB, D), lambda i: (0, i, 0)),
              pl.BlockSpec((2, RB, EF), lambda i: (0, i, 0)),
              pl.BlockSpec((D, D), lambda i: (0, 0)),
              pl.BlockSpec((D, D), lambda i: (0, 0))],
    out_specs=[pl.BlockSpec((RB, D), lambda i: (i, 0))],
    out_shape=[jax.ShapeDtypeStruct((N, D), jnp.float32)],
)

_update_ab_call = pl.pallas_call(
    _update_ab_body,
    grid=(N // RB,),
    in_specs=[pl.BlockSpec((RB, D), lambda i: (i, 0)),
              pl.BlockSpec((2, RB, D), lambda i: (0, i, 0)),
              pl.BlockSpec((2, RB, EF), lambda i: (0, i, 0)),
              pl.BlockSpec((D, D), lambda i: (0, 0)),
              pl.BlockSpec((D, D), lambda i: (0, 0)),
              pl.BlockSpec((D, D), lambda i: (0, 0)),
              pl.BlockSpec((D, D), lambda i: (0, 0))],
    out_specs=[pl.BlockSpec((RB, D), lambda i: (i, 0)),
               pl.BlockSpec((RB, D), lambda i: (i, 0)),
               pl.BlockSpec((RB, D), lambda i: (i, 0))],
    out_shape=[jax.ShapeDtypeStruct((N, D), jnp.float32),
               jax.ShapeDtypeStruct((N, D), jnp.float32),
               jax.ShapeDtypeStruct((N, D), jnp.float32)],
)


def _compact_body(src_hbm, dst_hbm, nt_hbm,
                  csrc_out, cdst_out, ceid_out, cnt_out, recv_out,
                  recv_sh, nt_v, srcb_v, dstb_v,
                  csrc_v, cdst_v, ceid_v, cdst2_v, ones_v, cnt_v):
    c = lax.axis_index("c")
    s = lax.axis_index("s")
    wid = c * 16 + s

    pltpu.sync_copy(nt_hbm, nt_v)

    zero16i = jnp.zeros((16,), jnp.int32)

    def _prefill(g, carry):
        sl = pl.ds(g * 16, 16)
        csrc_v[sl] = zero16i
        cdst_v[sl] = zero16i
        ceid_v[sl] = zero16i
        return carry

    lax.fori_loop(0, CAP // 16, _prefill, 0)

    def _zero_ones(r, carry):
        ones_v[r, :] = jnp.zeros((16,), jnp.float32)
        return carry

    lax.fori_loop(0, K, _zero_ones, 0)

    row0 = s * ROWS_PER_TILE
    for i in range(ROWS_PER_TILE // K):
        pltpu.sync_copy(ones_v, recv_sh.at[pl.ds(row0 + i * K, K)])
    _rem = ROWS_PER_TILE % K
    if _rem:
        pltpu.sync_copy(ones_v.at[pl.ds(0, _rem)],
                        recv_sh.at[pl.ds(row0 + (ROWS_PER_TILE // K) * K, _rem)])

    off = jnp.int32(0)
    for ib in range(EDGES_PER_TILE // SRCB):
        base = wid * EDGES_PER_TILE + ib * SRCB
        pltpu.sync_copy(src_hbm.at[pl.ds(base, SRCB)], srcb_v)
        pltpu.sync_copy(dst_hbm.at[pl.ds(base, SRCB)], dstb_v)

        def _grp(t, o, _base=base):
            sl = pl.ds(t * 16, 16)
            sv = srcb_v[sl]
            dv = dstb_v[sl]
            fl = plsc.load_gather(nt_v, [sv])
            msk = fl == TASK
            npos = plsc.all_reduce_population_count(msk)[0]
            plsc.store_compressed(csrc_v.at[pl.ds(o, 16)], sv, mask=msk)
            plsc.store_compressed(cdst_v.at[pl.ds(o, 16)], dv, mask=msk)
            eid = _base + t * 16 + lax.iota(jnp.int32, 16)
            plsc.store_compressed(ceid_v.at[pl.ds(o, 16)], eid, mask=msk)
            return o + npos

        off = lax.fori_loop(0, SRCB // 16, _grp, off)
    n = off

    def _c2(g, carry):
        sl = pl.ds(g * 16, 16)
        row = g // 3
        col = (g % 3) * 16
        cdst2_v[row, pl.ds(col, 16)] = cdst_v[sl]
        return carry

    lax.fori_loop(0, CAP // 16, _c2, 0)

    cnt_v[...] = jnp.broadcast_to(n, (16,))
    pltpu.sync_copy(cnt_v, cnt_out.at[wid])
    pltpu.sync_copy(csrc_v, csrc_out.at[wid])
    pltpu.sync_copy(cdst_v, cdst_out.at[wid])
    pltpu.sync_copy(ceid_v, ceid_out.at[wid])

    plsc.subcore_barrier()

    nch = (n + (K - 1)) // K

    def _rchunk(j, carry):
        def _fill(r, inner, _j=j):
            pos = _j * K + r
            ones_v[r, :] = jnp.where(pos < n,
                                     jnp.full((16,), 1.0, jnp.float32),
                                     jnp.zeros((16,), jnp.float32))
            return inner

        lax.fori_loop(0, K, _fill, 0)
        pltpu.sync_copy(ones_v, recv_sh.at[cdst2_v.at[j]], add=True)
        return carry

    lax.fori_loop(0, nch, _rchunk, 0)

    plsc.subcore_barrier()
    pltpu.sync_copy(recv_sh.at[pl.ds(row0, ROWS_PER_TILE)],
                    recv_out.at[c, pl.ds(row0, ROWS_PER_TILE)])


_compact_call = functools.partial(
    pl.kernel,
    out_type=[jax.ShapeDtypeStruct((NTILES, CAP), jnp.int32),
              jax.ShapeDtypeStruct((NTILES, CAP), jnp.int32),
              jax.ShapeDtypeStruct((NTILES, CAP), jnp.int32),
              jax.ShapeDtypeStruct((NTILES, 16), jnp.int32),
              jax.ShapeDtypeStruct((2, NPAD, EF), jnp.float32)],
    mesh=plsc.VectorSubcoreMesh(core_axis_name="c", subcore_axis_name="s"),
    compiler_params=_SC_PARAMS,
    scratch_types=[
        pltpu.VMEM_SHARED((NPAD, EF), jnp.float32),
        pltpu.VMEM((N,), jnp.int32),
        pltpu.VMEM((SRCB,), jnp.int32),
        pltpu.VMEM((SRCB,), jnp.int32),
        pltpu.VMEM((CAP,), jnp.int32),
        pltpu.VMEM((CAP,), jnp.int32),
        pltpu.VMEM((CAP,), jnp.int32),
        pltpu.VMEM((NCH, K), jnp.int32),
        pltpu.VMEM((K, EF), jnp.float32),
        pltpu.VMEM((16,), jnp.int32),
    ],
)(_compact_body)


def _edge_body(csrc_hbm, cdst_hbm, ceid_hbm, cnt_hbm, a_hbm, b_hbm, c_hbm,
               red_out,
               red_sh, csrc2_v, cdst2_v, ceid2_v, cnt_v, a_v, b_v, c_v,
               sem_a, sem_b, sem_c):
    c = lax.axis_index("c")
    s = lax.axis_index("s")
    wid = c * 16 + s

    pltpu.sync_copy(csrc_hbm.at[wid], csrc2_v)
    pltpu.sync_copy(cdst_hbm.at[wid], cdst2_v)
    pltpu.sync_copy(ceid_hbm.at[wid], ceid2_v)
    pltpu.sync_copy(cnt_hbm.at[wid], cnt_v)
    n = jnp.max(cnt_v[...])

    def _zero_a(r, carry):
        for j in range(D // 16):
            a_v[r, pl.ds(j * 16, 16)] = jnp.zeros((16,), jnp.float32)
        return carry

    lax.fori_loop(0, K, _zero_a, 0)

    row0 = s * ROWS_PER_TILE
    for i in range(ROWS_PER_TILE // K):
        pltpu.sync_copy(a_v, red_sh.at[pl.ds(row0 + i * K, K)])
    _rem = ROWS_PER_TILE % K
    if _rem:
        pltpu.sync_copy(a_v.at[pl.ds(0, _rem)],
                        red_sh.at[pl.ds(row0 + (ROWS_PER_TILE // K) * K, _rem)])

    plsc.subcore_barrier()

    nch = (n + (K - 1)) // K

    def _chunk(jc, carry):
        cpa = pltpu.async_copy(a_hbm.at[csrc2_v.at[jc]], a_v, sem_a)
        cpb = pltpu.async_copy(b_hbm.at[cdst2_v.at[jc]], b_v, sem_b)
        cpc = pltpu.async_copy(c_hbm.at[ceid2_v.at[jc]], c_v, sem_c)
        cpa.wait()
        cpb.wait()
        cpc.wait()

        def _compute_row(r, inner, _jc=jc):
            pos = _jc * K + r
            valid = pos < n
            for j in range(D // 16):
                sl2 = pl.ds(j * 16, 16)
                m = a_v[r, sl2] + b_v[r, sl2] + c_v[r, sl2]
                m = jnp.maximum(m, m * 0.01)
                a_v[r, sl2] = jnp.where(valid, m, jnp.zeros((16,), jnp.float32))
            return inner

        lax.fori_loop(0, K, _compute_row, 0)
        pltpu.sync_copy(a_v, red_sh.at[cdst2_v.at[jc]], add=True)
        return carry

    lax.fori_loop(0, nch, _chunk, 0)

    plsc.subcore_barrier()
    pltpu.sync_copy(red_sh.at[pl.ds(row0, ROWS_PER_TILE)],
                    red_out.at[c, pl.ds(row0, ROWS_PER_TILE)])


_edge_call = functools.partial(
    pl.kernel,
    out_type=[jax.ShapeDtypeStruct((2, NPAD, D), jnp.float32)],
    mesh=plsc.VectorSubcoreMesh(core_axis_name="c", subcore_axis_name="s"),
    compiler_params=_SC_PARAMS,
    scratch_types=[
        pltpu.VMEM_SHARED((NPAD, D), jnp.float32),
        pltpu.VMEM((NCH, K), jnp.int32),
        pltpu.VMEM((NCH, K), jnp.int32),
        pltpu.VMEM((NCH, K), jnp.int32),
        pltpu.VMEM((16,), jnp.int32),
        pltpu.VMEM((K, D), jnp.float32),
        pltpu.VMEM((K, D), jnp.float32),
        pltpu.VMEM((K, D), jnp.float32),
        pltpu.SemaphoreType.DMA,
        pltpu.SemaphoreType.DMA,
        pltpu.SemaphoreType.DMA,
    ],
)(_edge_body)


def kernel(nf, ef, edge_index, node_type, We0, Wn0, We1, Wn1):
    src = edge_index[0]
    dst = edge_index[1]
    a0, b0 = _ab_call(nf, We0[:D], We0[D:2 * D])
    eye8 = jnp.eye(8, dtype=jnp.float32)
    ef8 = ef.reshape(E // 8, 8 * EF)
    c0, c1 = _c_call(ef8, jnp.kron(eye8, We0[2 * D:]),
                     jnp.kron(eye8, We1[2 * D:]))
    c0 = c0.reshape(E, D)
    c1 = c1.reshape(E, D)
    csrc, cdst, ceid, cnt, recv = _compact_call(src, dst, node_type)
    csrc = csrc.reshape(NTILES, NCH, K)
    cdst = cdst.reshape(NTILES, NCH, K)
    ceid = ceid.reshape(NTILES, NCH, K)
    red0, = _edge_call(csrc, cdst, ceid, cnt, a0, b0, c0)
    h1, a1, b1 = _update_ab_call(nf, red0, recv, Wn0[:D], Wn0[D:],
                                 We1[:D], We1[D:2 * D])
    red1, = _edge_call(csrc, cdst, ceid, cnt, a1, b1, c1)
    h2, = _update_call(h1, red1, recv, Wn1[:D], Wn1[D:])
    return h2


# speedup vs baseline: 1.5947x; 1.5947x over previous
import functools

import jax
import jax.numpy as jnp
from jax import lax
from jax.experimental import pallas as pl
from jax.experimental.pallas import tpu as pltpu
from jax.experimental.pallas import tpu_sc as plsc

N = 10000
E = 320000
D = 128
EF = 16
TASK = 2

NTILES = 32
NPAD = 10016
ROWS_PER_TILE = NPAD // 16
EDGES_PER_TILE = E // NTILES
K = 48
CAP = 10080
NCH = CAP // K
SRCB = 2000
RB = 400
CB = 2560

_SC_PARAMS = pltpu.CompilerParams(use_tc_tiling_on_sc=False,
                                  needs_layout_passes=False)


def _c_body(eft_ref, w0_ref, w1_ref, c0_ref, c1_ref):
    e = eft_ref[...]
    dn = (((0,), (0,)), ((), ()))
    c0_ref[...] = lax.dot_general(e, w0_ref[...], dn,
                                  preferred_element_type=jnp.float32)
    c1_ref[...] = lax.dot_general(e, w1_ref[...], dn,
                                  preferred_element_type=jnp.float32)


_c_call = pl.pallas_call(
    _c_body,
    grid=(E // CB,),
    in_specs=[pl.BlockSpec((EF, CB), lambda i: (0, i)),
              pl.BlockSpec((EF, D), lambda i: (0, 0)),
              pl.BlockSpec((EF, D), lambda i: (0, 0))],
    out_specs=[pl.BlockSpec((CB, D), lambda i: (i, 0)),
               pl.BlockSpec((CB, D), lambda i: (i, 0))],
    out_shape=[jax.ShapeDtypeStruct((E, D), jnp.float32),
               jax.ShapeDtypeStruct((E, D), jnp.float32)],
)


def _ab_body(h_ref, wa_ref, wb_ref, a_ref, b_ref):
    h = h_ref[...]
    a_ref[...] = jnp.dot(h, wa_ref[...], preferred_element_type=jnp.float32)
    b_ref[...] = jnp.dot(h, wb_ref[...], preferred_element_type=jnp.float32)


_ab_call = pl.pallas_call(
    _ab_body,
    grid=(N // RB,),
    in_specs=[pl.BlockSpec((RB, D), lambda i: (i, 0)),
              pl.BlockSpec((D, D), lambda i: (0, 0)),
              pl.BlockSpec((D, D), lambda i: (0, 0))],
    out_specs=[pl.BlockSpec((RB, D), lambda i: (i, 0)),
               pl.BlockSpec((RB, D), lambda i: (i, 0))],
    out_shape=[jax.ShapeDtypeStruct((N, D), jnp.float32),
               jax.ShapeDtypeStruct((N, D), jnp.float32)],
)


def _node_update(h, red_ref, recv_ref, wh_ref, wr_ref):
    red = red_ref[0] + red_ref[1]
    x = (jnp.dot(h, wh_ref[...], preferred_element_type=jnp.float32)
         + jnp.dot(red, wr_ref[...], preferred_element_type=jnp.float32))
    o = jnp.maximum(x, x * 0.01)
    cnt = recv_ref[0, :, 0] + recv_ref[1, :, 0]
    return jnp.where((cnt > 0.0)[:, None], o, 0.0)


def _update_body(h_ref, red_ref, recv_ref, wh_ref, wr_ref, out_ref):
    out_ref[...] = _node_update(h_ref[...], red_ref, recv_ref, wh_ref, wr_ref)


def _update_ab_body(h_ref, red_ref, recv_ref, wh_ref, wr_ref, wa_ref, wb_ref,
                    out_ref, a_ref, b_ref):
    h1 = _node_update(h_ref[...], red_ref, recv_ref, wh_ref, wr_ref)
    out_ref[...] = h1
    a_ref[...] = jnp.dot(h1, wa_ref[...], preferred_element_type=jnp.float32)
    b_ref[...] = jnp.dot(h1, wb_ref[...], preferred_element_type=jnp.float32)


_update_call = pl.pallas_call(
    _update_body,
    grid=(N // RB,),
    in_specs=[pl.BlockSpec((RB, D), lambda i: (i, 0)),
              pl.BlockSpec((2, RB, D), lambda i: (0, i, 0)),
              pl.BlockSpec((2, RB, EF), lambda i: (0, i, 0)),
              pl.BlockSpec((D, D), lambda i: (0, 0)),
              pl.BlockSpec((D, D), lambda i: (0, 0))],
    out_specs=[pl.BlockSpec((RB, D), lambda i: (i, 0))],
    out_shape=[jax.ShapeDtypeStruct((N, D), jnp.float32)],
)

_update_ab_call = pl.pallas_call(
    _update_ab_body,
    grid=(N // RB,),
    in_specs=[pl.BlockSpec((RB, D), lambda i: (i, 0)),
              pl.BlockSpec((2, RB, D), lambda i: (0, i, 0)),
              pl.BlockSpec((2, RB, EF), lambda i: (0, i, 0)),
              pl.BlockSpec((D, D), lambda i: (0, 0)),
              pl.BlockSpec((D, D), lambda i: (0, 0)),
              pl.BlockSpec((D, D), lambda i: (0, 0)),
              pl.BlockSpec((D, D), lambda i: (0, 0))],
    out_specs=[pl.BlockSpec((RB, D), lambda i: (i, 0)),
               pl.BlockSpec((RB, D), lambda i: (i, 0)),
               pl.BlockSpec((RB, D), lambda i: (i, 0))],
    out_shape=[jax.ShapeDtypeStruct((N, D), jnp.float32),
               jax.ShapeDtypeStruct((N, D), jnp.float32),
               jax.ShapeDtypeStruct((N, D), jnp.float32)],
)


def _compact_body(src_hbm, dst_hbm, nt_hbm,
                  csrc_out, cdst_out, ceid_out, cnt_out, recv_out,
                  recv_sh, nt_v, srcb_v, dstb_v,
                  csrc_v, cdst_v, ceid_v, cdst2_v, ones_v, cnt_v):
    c = lax.axis_index("c")
    s = lax.axis_index("s")
    wid = c * 16 + s

    pltpu.sync_copy(nt_hbm, nt_v)

    zero16i = jnp.zeros((16,), jnp.int32)

    def _prefill(g, carry):
        sl = pl.ds(g * 16, 16)
        csrc_v[sl] = zero16i
        cdst_v[sl] = zero16i
        ceid_v[sl] = zero16i
        return carry

    lax.fori_loop(0, CAP // 16, _prefill, 0)

    def _zero_ones(r, carry):
        ones_v[r, :] = jnp.zeros((16,), jnp.float32)
        return carry

    lax.fori_loop(0, K, _zero_ones, 0)

    row0 = s * ROWS_PER_TILE
    for i in range(ROWS_PER_TILE // K):
        pltpu.sync_copy(ones_v, recv_sh.at[pl.ds(row0 + i * K, K)])
    _rem = ROWS_PER_TILE % K
    if _rem:
        pltpu.sync_copy(ones_v.at[pl.ds(0, _rem)],
                        recv_sh.at[pl.ds(row0 + (ROWS_PER_TILE // K) * K, _rem)])

    off = jnp.int32(0)
    for ib in range(EDGES_PER_TILE // SRCB):
        base = wid * EDGES_PER_TILE + ib * SRCB
        pltpu.sync_copy(src_hbm.at[pl.ds(base, SRCB)], srcb_v)
        pltpu.sync_copy(dst_hbm.at[pl.ds(base, SRCB)], dstb_v)

        def _grp(t, o, _base=base):
            sl = pl.ds(t * 16, 16)
            sv = srcb_v[sl]
            dv = dstb_v[sl]
            fl = plsc.load_gather(nt_v, [sv])
            msk = fl == TASK
            npos = plsc.all_reduce_population_count(msk)[0]
            plsc.store_compressed(csrc_v.at[pl.ds(o, 16)], sv, mask=msk)
            plsc.store_compressed(cdst_v.at[pl.ds(o, 16)], dv, mask=msk)
            eid = _base + t * 16 + lax.iota(jnp.int32, 16)
            plsc.store_compressed(ceid_v.at[pl.ds(o, 16)], eid, mask=msk)
            return o + npos

        off = lax.fori_loop(0, SRCB // 16, _grp, off)
    n = off

    def _c2(g, carry):
        sl = pl.ds(g * 16, 16)
        row = g // 3
        col = (g % 3) * 16
        cdst2_v[row, pl.ds(col, 16)] = cdst_v[sl]
        return carry

    lax.fori_loop(0, CAP // 16, _c2, 0)

    cnt_v[...] = jnp.broadcast_to(n, (16,))
    pltpu.sync_copy(cnt_v, cnt_out.at[wid])
    pltpu.sync_copy(csrc_v, csrc_out.at[wid])
    pltpu.sync_copy(cdst_v, cdst_out.at[wid])
    pltpu.sync_copy(ceid_v, ceid_out.at[wid])

    plsc.subcore_barrier()

    nch = (n + (K - 1)) // K

    def _rchunk(j, carry):
        def _fill(r, inner, _j=j):
            pos = _j * K + r
            ones_v[r, :] = jnp.where(pos < n,
                                     jnp.full((16,), 1.0, jnp.float32),
                                     jnp.zeros((16,), jnp.float32))
            return inner

        lax.fori_loop(0, K, _fill, 0)
        pltpu.sync_copy(ones_v, recv_sh.at[cdst2_v.at[j]], add=True)
        return carry

    lax.fori_loop(0, nch, _rchunk, 0)

    plsc.subcore_barrier()
    pltpu.sync_copy(recv_sh.at[pl.ds(row0, ROWS_PER_TILE)],
                    recv_out.at[c, pl.ds(row0, ROWS_PER_TILE)])


_compact_call = functools.partial(
    pl.kernel,
    out_type=[jax.ShapeDtypeStruct((NTILES, CAP), jnp.int32),
              jax.ShapeDtypeStruct((NTILES, CAP), jnp.int32),
              jax.ShapeDtypeStruct((NTILES, CAP), jnp.int32),
              jax.ShapeDtypeStruct((NTILES, 16), jnp.int32),
              jax.ShapeDtypeStruct((2, NPAD, EF), jnp.float32)],
    mesh=plsc.VectorSubcoreMesh(core_axis_name="c", subcore_axis_name="s"),
    compiler_params=_SC_PARAMS,
    scratch_types=[
        pltpu.VMEM_SHARED((NPAD, EF), jnp.float32),
        pltpu.VMEM((N,), jnp.int32),
        pltpu.VMEM((SRCB,), jnp.int32),
        pltpu.VMEM((SRCB,), jnp.int32),
        pltpu.VMEM((CAP,), jnp.int32),
        pltpu.VMEM((CAP,), jnp.int32),
        pltpu.VMEM((CAP,), jnp.int32),
        pltpu.VMEM((NCH, K), jnp.int32),
        pltpu.VMEM((K, EF), jnp.float32),
        pltpu.VMEM((16,), jnp.int32),
    ],
)(_compact_body)


def _edge_body(csrc_hbm, cdst_hbm, ceid_hbm, cnt_hbm, a_hbm, b_hbm, c_hbm,
               red_out,
               red_sh, csrc2_v, cdst2_v, ceid2_v, cnt_v, a_v, b_v, c_v,
               sem_a, sem_b, sem_c):
    c = lax.axis_index("c")
    s = lax.axis_index("s")
    wid = c * 16 + s

    pltpu.sync_copy(csrc_hbm.at[wid], csrc2_v)
    pltpu.sync_copy(cdst_hbm.at[wid], cdst2_v)
    pltpu.sync_copy(ceid_hbm.at[wid], ceid2_v)
    pltpu.sync_copy(cnt_hbm.at[wid], cnt_v)
    n = jnp.max(cnt_v[...])

    def _zero_a(r, carry):
        for j in range(D // 16):
            a_v[r, pl.ds(j * 16, 16)] = jnp.zeros((16,), jnp.float32)
        return carry

    lax.fori_loop(0, K, _zero_a, 0)

    row0 = s * ROWS_PER_TILE
    for i in range(ROWS_PER_TILE // K):
        pltpu.sync_copy(a_v, red_sh.at[pl.ds(row0 + i * K, K)])
    _rem = ROWS_PER_TILE % K
    if _rem:
        pltpu.sync_copy(a_v.at[pl.ds(0, _rem)],
                        red_sh.at[pl.ds(row0 + (ROWS_PER_TILE // K) * K, _rem)])

    plsc.subcore_barrier()

    nch = (n + (K - 1)) // K

    def _chunk(jc, carry):
        cpa = pltpu.async_copy(a_hbm.at[csrc2_v.at[jc]], a_v, sem_a)
        cpb = pltpu.async_copy(b_hbm.at[cdst2_v.at[jc]], b_v, sem_b)
        cpc = pltpu.async_copy(c_hbm.at[ceid2_v.at[jc]], c_v, sem_c)
        cpa.wait()
        cpb.wait()
        cpc.wait()

        def _compute_row(r, inner, _jc=jc):
            pos = _jc * K + r
            valid = pos < n
            for j in range(D // 16):
                sl2 = pl.ds(j * 16, 16)
                m = a_v[r, sl2] + b_v[r, sl2] + c_v[r, sl2]
                m = jnp.maximum(m, m * 0.01)
                a_v[r, sl2] = jnp.where(valid, m, jnp.zeros((16,), jnp.float32))
            return inner

        lax.fori_loop(0, K, _compute_row, 0)
        pltpu.sync_copy(a_v, red_sh.at[cdst2_v.at[jc]], add=True)
        return carry

    lax.fori_loop(0, nch, _chunk, 0)

    plsc.subcore_barrier()
    pltpu.sync_copy(red_sh.at[pl.ds(row0, ROWS_PER_TILE)],
                    red_out.at[c, pl.ds(row0, ROWS_PER_TILE)])


_edge_call = functools.partial(
    pl.kernel,
    out_type=[jax.ShapeDtypeStruct((2, NPAD, D), jnp.float32)],
    mesh=plsc.VectorSubcoreMesh(core_axis_name="c", subcore_axis_name="s"),
    compiler_params=_SC_PARAMS,
    scratch_types=[
        pltpu.VMEM_SHARED((NPAD, D), jnp.float32),
        pltpu.VMEM((NCH, K), jnp.int32),
        pltpu.VMEM((NCH, K), jnp.int32),
        pltpu.VMEM((NCH, K), jnp.int32),
        pltpu.VMEM((16,), jnp.int32),
        pltpu.VMEM((K, D), jnp.float32),
        pltpu.VMEM((K, D), jnp.float32),
        pltpu.VMEM((K, D), jnp.float32),
        pltpu.SemaphoreType.DMA,
        pltpu.SemaphoreType.DMA,
        pltpu.SemaphoreType.DMA,
    ],
)(_edge_body)


def kernel(nf, ef, edge_index, node_type, We0, Wn0, We1, Wn1):
    src = edge_index[0]
    dst = edge_index[1]
    a0, b0 = _ab_call(nf, We0[:D], We0[D:2 * D])
    c0, c1 = _c_call(ef.T, We0[2 * D:], We1[2 * D:])
    csrc, cdst, ceid, cnt, recv = _compact_call(src, dst, node_type)
    csrc = csrc.reshape(NTILES, NCH, K)
    cdst = cdst.reshape(NTILES, NCH, K)
    ceid = ceid.reshape(NTILES, NCH, K)
    red0, = _edge_call(csrc, cdst, ceid, cnt, a0, b0, c0)
    h1, a1, b1 = _update_ab_call(nf, red0, recv, Wn0[:D], Wn0[D:],
                                 We1[:D], We1[D:2 * D])
    red1, = _edge_call(csrc, cdst, ceid, cnt, a1, b1, c1)
    h2, = _update_call(h1, red1, recv, Wn1[:D], Wn1[D:])
    return h2
